# Initial kernel scaffold; baseline (speedup 1.0000x reference)
import jax
import jax.numpy as jnp
from jax.experimental import pallas as pl


def kernel(x, norm_weight, gate_weight, gate_bias, mlp1_weight, mlp1_bias, mlp2_weight, mlp2_bias):
    raise NotImplementedError("write your pallas kernel here")


# speedup vs baseline: 6.5463x; 6.5463x over previous
import jax
import jax.numpy as jnp
from jax.experimental import pallas as pl
from jax.experimental.pallas import tpu as pltpu

_B, _S = 1, 128
_HID = 768
_INTER = 768
_NEXP = 16
_LIMIT = 7.0
_EPS = 1e-05
_ALPHA = 1.702


def _moe_kernel(x_ref, nw_ref, gw_ref, gb_ref,
                w1g_ref, w1l_ref, b1g_ref, b1l_ref,
                w2_ref, b2_ref,
                out_ref,
                normed_ref, i1_ref, i2_ref, wa_ref, wb_ref, acc_ref):
    e = pl.program_id(0)

    @pl.when(e == 0)
    def _prologue():
        xf = x_ref[0]
        rms = jnp.mean(jnp.square(xf), axis=-1, keepdims=True)
        normed_f = xf * jax.lax.rsqrt(rms + jnp.float32(_EPS))
        normed_f = normed_f * nw_ref[...].astype(jnp.float32)
        normed = normed_f.astype(jnp.bfloat16)
        normed_ref[...] = normed
        gl = jax.lax.dot_general(
            normed, gw_ref[...],
            dimension_numbers=(((1,), (1,)), ((), ())),
            preferred_element_type=jnp.float32)
        logits = gl + gb_ref[...].astype(jnp.float32)
        lane = jax.lax.broadcasted_iota(jnp.int32, logits.shape, 1)
        m1 = jnp.max(logits, axis=1, keepdims=True)
        i1 = jnp.min(jnp.where(logits == m1, lane, _NEXP), axis=1,
                     keepdims=True)
        masked = jnp.where(lane == i1, -jnp.inf, logits)
        m2 = jnp.max(masked, axis=1, keepdims=True)
        i2 = jnp.min(jnp.where(masked == m2, lane, _NEXP), axis=1,
                     keepdims=True)
        eb = jnp.exp(m2 - m1)
        denom = 1.0 + eb
        i1_ref[...] = i1
        i2_ref[...] = i2
        wa_ref[...] = 1.0 / denom
        wb_ref[...] = eb / denom
        acc_ref[...] = jnp.zeros_like(acc_ref)

    normed = normed_ref[...]
    hg = jax.lax.dot_general(
        normed, w1g_ref[0],
        dimension_numbers=(((1,), (1,)), ((), ())),
        preferred_element_type=jnp.float32)
    hl = jax.lax.dot_general(
        normed, w1l_ref[0],
        dimension_numbers=(((1,), (1,)), ((), ())),
        preferred_element_type=jnp.float32)
    hg = hg + b1g_ref[0].astype(jnp.float32)
    hl = hl + b1l_ref[0].astype(jnp.float32)
    hg = jnp.minimum(hg, _LIMIT)
    hl = jnp.clip(hl, -_LIMIT, _LIMIT)
    act = hg * jax.nn.sigmoid(_ALPHA * hg) * (hl + 1.0)
    y = jax.lax.dot_general(
        act.astype(jnp.bfloat16), w2_ref[0],
        dimension_numbers=(((1,), (1,)), ((), ())),
        preferred_element_type=jnp.float32)
    y = y + b2_ref[0].astype(jnp.float32)
    coef = (jnp.where(i1_ref[...] == e, wa_ref[...], 0.0)
            + jnp.where(i2_ref[...] == e, wb_ref[...], 0.0))
    acc_ref[...] += coef * y

    @pl.when(e == _NEXP - 1)
    def _epilogue():
        xc = x_ref[0].astype(jnp.bfloat16)
        out_ref[0] = xc + acc_ref[...].astype(jnp.bfloat16)


def kernel(x, norm_weight, gate_weight, gate_bias, mlp1_weight, mlp1_bias,
           mlp2_weight, mlp2_bias):
    mlp1_r = mlp1_weight.reshape(_NEXP, _INTER, 2 * _HID)
    b1 = mlp1_bias.reshape(_NEXP, _INTER, 2)
    b1g = b1[:, :, 0].reshape(_NEXP, 1, _INTER)
    b1l = b1[:, :, 1].reshape(_NEXP, 1, _INTER)
    b2 = mlp2_bias.reshape(_NEXP, 1, _HID)
    nw = norm_weight.reshape(1, _HID)
    gb = gate_bias.reshape(1, _NEXP)

    grid = (_NEXP,)
    out = pl.pallas_call(
        _moe_kernel,
        grid=grid,
        in_specs=[
            pl.BlockSpec((_B, _S, _HID), lambda e: (0, 0, 0)),
            pl.BlockSpec((1, _HID), lambda e: (0, 0)),
            pl.BlockSpec((_NEXP, _HID), lambda e: (0, 0)),
            pl.BlockSpec((1, _NEXP), lambda e: (0, 0)),
            pl.BlockSpec((1, _INTER, _HID), lambda e: (e, 0, 0)),
            pl.BlockSpec((1, _INTER, _HID), lambda e: (e, 0, 1)),
            pl.BlockSpec((1, 1, _INTER), lambda e: (e, 0, 0)),
            pl.BlockSpec((1, 1, _INTER), lambda e: (e, 0, 0)),
            pl.BlockSpec((1, _HID, _INTER), lambda e: (e, 0, 0)),
            pl.BlockSpec((1, 1, _HID), lambda e: (e, 0, 0)),
        ],
        out_specs=pl.BlockSpec((_B, _S, _HID), lambda e: (0, 0, 0)),
        out_shape=jax.ShapeDtypeStruct((_B, _S, _HID), jnp.bfloat16),
        scratch_shapes=[
            pltpu.VMEM((_S, _HID), jnp.bfloat16),
            pltpu.VMEM((_S, 1), jnp.int32),
            pltpu.VMEM((_S, 1), jnp.int32),
            pltpu.VMEM((_S, 1), jnp.float32),
            pltpu.VMEM((_S, 1), jnp.float32),
            pltpu.VMEM((_S, _HID), jnp.float32),
        ],
        compiler_params=pltpu.CompilerParams(
            dimension_semantics=("arbitrary",)),
    )(x, nw, gate_weight, gb, mlp1_r, mlp1_r, b1g, b1l, mlp2_weight, b2)
    return out


# speedup vs baseline: 12.0304x; 1.8377x over previous
import jax
import jax.numpy as jnp
from jax.experimental import pallas as pl
from jax.experimental.pallas import tpu as pltpu

_B, _S = 1, 128
_HID = 768
_INTER = 768
_NEXP = 16
_LIMIT = 7.0
_EPS = 1e-05
_ALPHA = 1.702


def _moe_kernel(x_ref, nw_ref, gw_ref, gb_ref,
                w1_ref, b1g_ref, b1l_ref,
                w2_ref, b2_ref,
                out_ref,
                normedT_ref, i1_ref, i2_ref, wa_ref, wb_ref,
                acc_ref, h_ref):
    e = pl.program_id(0)

    @pl.when(e == 0)
    def _prologue():
        xf = x_ref[0]
        rms = jnp.mean(jnp.square(xf), axis=-1, keepdims=True)
        normed_f = xf * jax.lax.rsqrt(rms + jnp.float32(_EPS))
        normed_f = normed_f * nw_ref[...].astype(jnp.float32)
        normed = normed_f.astype(jnp.bfloat16)
        normedT_ref[...] = normed.T
        gl = jax.lax.dot_general(
            normed, gw_ref[...],
            dimension_numbers=(((1,), (1,)), ((), ())),
            preferred_element_type=jnp.float32)
        logits = gl + gb_ref[...].astype(jnp.float32)
        logitsT = logits.T
        row = jax.lax.broadcasted_iota(jnp.int32, logitsT.shape, 0)
        m1 = jnp.max(logitsT, axis=0, keepdims=True)
        i1 = jnp.min(jnp.where(logitsT == m1, row, _NEXP), axis=0,
                     keepdims=True)
        masked = jnp.where(row == i1, -jnp.inf, logitsT)
        m2 = jnp.max(masked, axis=0, keepdims=True)
        i2 = jnp.min(jnp.where(masked == m2, row, _NEXP), axis=0,
                     keepdims=True)
        eb = jnp.exp(m2 - m1)
        denom = 1.0 + eb
        i1_ref[...] = i1
        i2_ref[...] = i2
        wa_ref[...] = 1.0 / denom
        wb_ref[...] = eb / denom
        acc_ref[...] = jnp.zeros_like(acc_ref)

    normedT = normedT_ref[...]
    h_ref[...] = jax.lax.dot_general(
        w1_ref[0], normedT,
        dimension_numbers=(((1,), (0,)), ((), ())),
        preferred_element_type=jnp.float32)
    hg = h_ref[pl.Slice(0, _INTER, 2), :]
    hl = h_ref[pl.Slice(1, _INTER, 2), :]
    hg = hg + b1g_ref[0].astype(jnp.float32)
    hl = hl + b1l_ref[0].astype(jnp.float32)
    hg = jnp.minimum(hg, _LIMIT)
    hl = jnp.clip(hl, -_LIMIT, _LIMIT)
    act = hg * jax.nn.sigmoid(_ALPHA * hg) * (hl + 1.0)
    y = jax.lax.dot_general(
        w2_ref[0], act.astype(jnp.bfloat16),
        dimension_numbers=(((1,), (0,)), ((), ())),
        preferred_element_type=jnp.float32)
    y = y + b2_ref[0].astype(jnp.float32)
    coef = (jnp.where(i1_ref[...] == e, wa_ref[...], 0.0)
            + jnp.where(i2_ref[...] == e, wb_ref[...], 0.0))
    acc_ref[...] += coef * y

    @pl.when(e == _NEXP - 1)
    def _epilogue():
        xc = x_ref[0].astype(jnp.bfloat16)
        mixed = acc_ref[...].T
        out_ref[0] = xc + mixed.astype(jnp.bfloat16)


def kernel(x, norm_weight, gate_weight, gate_bias, mlp1_weight, mlp1_bias,
           mlp2_weight, mlp2_bias):
    b1 = mlp1_bias.reshape(_NEXP, _INTER, 2)
    b1g = b1[:, :, 0].reshape(_NEXP, _INTER, 1)
    b1l = b1[:, :, 1].reshape(_NEXP, _INTER, 1)
    b2 = mlp2_bias.reshape(_NEXP, _HID, 1)
    nw = norm_weight.reshape(1, _HID)
    gb = gate_bias.reshape(1, _NEXP)

    grid = (_NEXP,)
    out = pl.pallas_call(
        _moe_kernel,
        grid=grid,
        in_specs=[
            pl.BlockSpec((_B, _S, _HID), lambda e: (0, 0, 0)),
            pl.BlockSpec((1, _HID), lambda e: (0, 0)),
            pl.BlockSpec((_NEXP, _HID), lambda e: (0, 0)),
            pl.BlockSpec((1, _NEXP), lambda e: (0, 0)),
            pl.BlockSpec((1, 2 * _INTER, _HID), lambda e: (e, 0, 0)),
            pl.BlockSpec((1, _INTER, 1), lambda e: (e, 0, 0)),
            pl.BlockSpec((1, _INTER, 1), lambda e: (e, 0, 0)),
            pl.BlockSpec((1, _HID, _INTER), lambda e: (e, 0, 0)),
            pl.BlockSpec((1, _HID, 1), lambda e: (e, 0, 0)),
        ],
        out_specs=pl.BlockSpec((_B, _S, _HID), lambda e: (0, 0, 0)),
        out_shape=jax.ShapeDtypeStruct((_B, _S, _HID), jnp.bfloat16),
        scratch_shapes=[
            pltpu.VMEM((_HID, _S), jnp.bfloat16),
            pltpu.VMEM((1, _S), jnp.int32),
            pltpu.VMEM((1, _S), jnp.int32),
            pltpu.VMEM((1, _S), jnp.float32),
            pltpu.VMEM((1, _S), jnp.float32),
            pltpu.VMEM((_HID, _S), jnp.float32),
            pltpu.VMEM((2 * _INTER, _S), jnp.float32),
        ],
        compiler_params=pltpu.CompilerParams(
            dimension_semantics=("arbitrary",)),
    )(x, nw, gate_weight, gb, mlp1_weight, b1g, b1l, mlp2_weight, b2)
    return out


# speedup vs baseline: 18.3736x; 1.5273x over previous
import jax
import jax.numpy as jnp
from jax.experimental import pallas as pl
from jax.experimental.pallas import tpu as pltpu

_B, _S = 1, 128
_HID = 768
_INTER = 768
_NEXP = 16
_LIMIT = 7.0
_EPS = 1e-05
_ALPHA = 1.702


def _moe_kernel(x_ref, nw_ref, gw_ref, gb_ref,
                w1_ref, b1_ref, w2_ref, b2_ref,
                out_ref,
                normedT_ref, c_ref, b1t_ref, acc_ref, h_ref):
    e = pl.program_id(0)

    @pl.when(e == 0)
    def _prologue():
        xf = x_ref[0]
        rms = jnp.mean(jnp.square(xf), axis=-1, keepdims=True)
        normed_f = xf * jax.lax.rsqrt(rms + jnp.float32(_EPS))
        normed_f = normed_f * nw_ref[...].astype(jnp.float32)
        normed = normed_f.astype(jnp.bfloat16)
        normedT_ref[...] = normed.T
        gl = jax.lax.dot_general(
            normed, gw_ref[...],
            dimension_numbers=(((1,), (1,)), ((), ())),
            preferred_element_type=jnp.float32)
        logits = gl + gb_ref[...].astype(jnp.float32)
        logitsT = logits.T
        row = jax.lax.broadcasted_iota(jnp.int32, logitsT.shape, 0)
        m1 = jnp.max(logitsT, axis=0, keepdims=True)
        i1 = jnp.min(jnp.where(logitsT == m1, row, _NEXP), axis=0,
                     keepdims=True)
        masked = jnp.where(row == i1, -jnp.inf, logitsT)
        m2 = jnp.max(masked, axis=0, keepdims=True)
        i2 = jnp.min(jnp.where(masked == m2, row, _NEXP), axis=0,
                     keepdims=True)
        eb = jnp.exp(m2 - m1)
        denom = 1.0 + eb
        wa = 1.0 / denom
        wb = eb / denom
        c_ref[...] = (jnp.where(row == i1, wa, 0.0)
                      + jnp.where(row == i2, wb, 0.0))
        b1t_ref[...] = b1_ref[...].astype(jnp.float32).T
        acc_ref[...] = jnp.zeros_like(acc_ref)

    normedT = normedT_ref[...]
    h_ref[...] = jax.lax.dot_general(
        w1_ref[0], normedT,
        dimension_numbers=(((1,), (0,)), ((), ())),
        preferred_element_type=jnp.float32)
    hg = h_ref[pl.Slice(0, _INTER, 2), :]
    hl = h_ref[pl.Slice(1, _INTER, 2), :]
    lane = jax.lax.broadcasted_iota(jnp.int32, (_INTER, _NEXP), 1)
    bg = b1t_ref[pl.Slice(0, _INTER, 2), :]
    bl = b1t_ref[pl.Slice(1, _INTER, 2), :]
    hg = hg + jnp.sum(jnp.where(lane == e, bg, 0.0), axis=1, keepdims=True)
    hl = hl + jnp.sum(jnp.where(lane == e, bl, 0.0), axis=1, keepdims=True)
    hg = jnp.minimum(hg, _LIMIT)
    hl = jnp.clip(hl, -_LIMIT, _LIMIT)
    act = hg * jax.nn.sigmoid(_ALPHA * hg) * (hl + 1.0)
    y = jax.lax.dot_general(
        w2_ref[0], act.astype(jnp.bfloat16),
        dimension_numbers=(((1,), (0,)), ((), ())),
        preferred_element_type=jnp.float32)
    acc_ref[...] += c_ref[pl.ds(e, 1), :] * y

    @pl.when(e == _NEXP - 1)
    def _epilogue():
        b2c = jax.lax.dot_general(
            b2_ref[...].astype(jnp.float32), c_ref[...],
            dimension_numbers=(((0,), (0,)), ((), ())),
            precision=jax.lax.Precision.HIGHEST,
            preferred_element_type=jnp.float32)
        xc = x_ref[0].astype(jnp.bfloat16)
        mixed = (acc_ref[...] + b2c).T
        out_ref[0] = xc + mixed.astype(jnp.bfloat16)


def kernel(x, norm_weight, gate_weight, gate_bias, mlp1_weight, mlp1_bias,
           mlp2_weight, mlp2_bias):
    nw = norm_weight.reshape(1, _HID)
    gb = gate_bias.reshape(1, _NEXP)

    grid = (_NEXP,)
    out = pl.pallas_call(
        _moe_kernel,
        grid=grid,
        in_specs=[
            pl.BlockSpec((_B, _S, _HID), lambda e: (0, 0, 0)),
            pl.BlockSpec((1, _HID), lambda e: (0, 0)),
            pl.BlockSpec((_NEXP, _HID), lambda e: (0, 0)),
            pl.BlockSpec((1, _NEXP), lambda e: (0, 0)),
            pl.BlockSpec((1, 2 * _INTER, _HID), lambda e: (e, 0, 0)),
            pl.BlockSpec((_NEXP, 2 * _INTER), lambda e: (0, 0)),
            pl.BlockSpec((1, _HID, _INTER), lambda e: (e, 0, 0)),
            pl.BlockSpec((_NEXP, _HID), lambda e: (0, 0)),
        ],
        out_specs=pl.BlockSpec((_B, _S, _HID), lambda e: (0, 0, 0)),
        out_shape=jax.ShapeDtypeStruct((_B, _S, _HID), jnp.bfloat16),
        scratch_shapes=[
            pltpu.VMEM((_HID, _S), jnp.bfloat16),
            pltpu.VMEM((_NEXP, _S), jnp.float32),
            pltpu.VMEM((2 * _INTER, _NEXP), jnp.float32),
            pltpu.VMEM((_HID, _S), jnp.float32),
            pltpu.VMEM((2 * _INTER, _S), jnp.float32),
        ],
        compiler_params=pltpu.CompilerParams(
            dimension_semantics=("arbitrary",)),
    )(x, nw, gate_weight, gb, mlp1_weight, mlp1_bias, mlp2_weight, mlp2_bias)
    return out
